# baseline (speedup 1.0000x reference)
import jax
import jax.numpy as jnp
from jax import lax
from jax.experimental import pallas as pl
from jax.experimental.pallas import tpu as pltpu
from jax.experimental.pallas import tpu_sc as plsc

NC = 2
NS = 16
NW = NC * NS

G1 = 128
RING = 4
LOOKAHEAD = 2


def _make_sc_call(batch, seq, embed, vocab):
    bpw = batch // NW
    assert bpw % RING == 0
    g2 = seq - G1

    def body(idx_hbm, table_hbm, pe_hbm, out_hbm, idx_v, pe_v,
             r0, r1, r2, r3, g0, g1, g2s, g3, s0, s1, s2, s3):
        rows = (r0, r1, r2, r3)
        gsem = (g0, g1, g2s, g3)
        ssem = (s0, s1, s2, s3)
        cid = lax.axis_index("c")
        sid = lax.axis_index("s")
        wid = sid * NC + cid
        base = wid * bpw

        pltpu.sync_copy(idx_hbm.at[pl.ds(base, bpw)], idx_v)
        pltpu.sync_copy(pe_hbm, pe_v)

        def gathers(t, b):
            return (
                pltpu.make_async_copy(
                    table_hbm.at[idx_v.at[t, pl.ds(0, G1)]],
                    rows[b].at[pl.ds(0, G1)], gsem[b]),
                pltpu.make_async_copy(
                    table_hbm.at[idx_v.at[t, pl.ds(G1, g2)]],
                    rows[b].at[pl.ds(G1, g2)], gsem[b]),
            )

        def start_gather(t, b):
            ga, gb = gathers(t, b)
            ga.start()
            gb.start()

        def wait_gather(t, b):
            ga, gb = gathers(t, b)
            ga.wait()
            gb.wait()

        def store(t, b):
            return pltpu.make_async_copy(rows[b], out_hbm.at[base + t], ssem[b])

        start_gather(0, 0)
        start_gather(1, 1)

        nvec = embed // 16

        def outer(t0, carry):
            for b in range(RING):
                t = t0 * RING + b
                wait_gather(t, b)

                def row_body(r, _, b=b):
                    for k in range(nvec):
                        sl = pl.ds(k * 16, 16)
                        rows[b][r, sl] = rows[b][r, sl] + pe_v[r, sl]
                    return 0

                lax.fori_loop(0, seq, row_body, 0, unroll=4)
                store(t, b).start()

                tn = t + LOOKAHEAD
                bn = (b + LOOKAHEAD) % RING

                @pl.when(tn < bpw)
                def _(tn=tn, bn=bn):
                    @pl.when(tn >= RING)
                    def _():
                        store(tn - RING, bn).wait()
                    start_gather(tn, bn)
            return carry

        lax.fori_loop(0, bpw // RING, outer, 0)

        for b in range(RING):
            store(bpw - RING + b, b).wait()

    return pl.kernel(
        body,
        out_type=jax.ShapeDtypeStruct((batch, seq, embed), jnp.float32),
        mesh=plsc.VectorSubcoreMesh(core_axis_name="c", subcore_axis_name="s"),
        compiler_params=pltpu.CompilerParams(use_tc_tiling_on_sc=False),
        scratch_types=[
            pltpu.VMEM((bpw, seq), jnp.int32),
            pltpu.VMEM((seq, embed), jnp.float32),
            pltpu.VMEM((seq, embed), jnp.float32),
            pltpu.VMEM((seq, embed), jnp.float32),
            pltpu.VMEM((seq, embed), jnp.float32),
            pltpu.VMEM((seq, embed), jnp.float32),
            pltpu.SemaphoreType.DMA,
            pltpu.SemaphoreType.DMA,
            pltpu.SemaphoreType.DMA,
            pltpu.SemaphoreType.DMA,
            pltpu.SemaphoreType.DMA,
            pltpu.SemaphoreType.DMA,
            pltpu.SemaphoreType.DMA,
            pltpu.SemaphoreType.DMA,
        ],
    )


def _pos_encoding(seq_len, d_model):
    pos = jnp.arange(1, 1 + seq_len, dtype=jnp.float32)
    power = jnp.arange(0, d_model, 2, dtype=jnp.float32) / d_model
    divisor = jnp.power(10000.0, power)
    angles = pos[:, None] / divisor[None, :]
    return jnp.stack([jnp.sin(angles), jnp.cos(angles)], axis=-1).reshape(
        seq_len, d_model)


def kernel(inputs, table):
    batch, seq = inputs.shape
    vocab, embed = table.shape
    assert batch % NW == 0

    pe = _pos_encoding(seq, embed)
    call = _make_sc_call(batch, seq, embed, vocab)
    return call(inputs, table, pe)
